# baseline (speedup 1.0000x reference)
import functools

import jax
import jax.numpy as jnp
from jax.experimental import pallas as pl
from jax.experimental.pallas import tpu as pltpu


def _tap_stack(v, H, W, taps_ref):
    C, L = v.shape
    l = jax.lax.broadcasted_iota(jnp.int32, (1, L), 1)
    wpos = jax.lax.rem(l, W)
    for kh in range(3):
        for kw in range(3):
            delta = W * (kh - 1) + (kw - 1)
            t = jnp.roll(v, -delta, axis=1) if delta else v
            mask = None
            if kw == 0:
                mask = wpos != 0
            elif kw == 2:
                mask = wpos != W - 1
            if kh == 0:
                mh = l >= W
                mask = mh if mask is None else (mask & mh)
            elif kh == 2:
                mh = l < L - W
                mask = mh if mask is None else (mask & mh)
            if mask is not None:
                t = jnp.where(mask, t, jnp.bfloat16(0))
            taps_ref[C * (3 * kh + kw):C * (3 * kh + kw + 1), :] = t


def _planar_kernel(x_ref, w1_ref, w2_ref, b_ref, o_ref, t_ref, *, B, H, W):
    b1c = b_ref[:, 0:1]
    b2c = b_ref[:, 1:2]
    w1v = w1_ref[...]
    w2v = w2_ref[...]
    for b in range(B):
        xb = x_ref[b]
        _tap_stack(xb, H, W, t_ref)
        a1 = jnp.dot(w1v, t_ref[...], preferred_element_type=jnp.float32)
        h1 = jnp.maximum(a1 + b1c, 0.0).astype(jnp.bfloat16)
        _tap_stack(h1, H, W, t_ref)
        a2 = jnp.dot(w2v, t_ref[...], preferred_element_type=jnp.float32)
        o_ref[b] = jnp.maximum(a2 + b2c, 0.0)


def kernel(x_nchw, w1, b1, w2, b2):
    N, C, H, W = x_nchw.shape
    B = 16 if N % 16 == 0 else 8
    L = H * W

    xv = x_nchw.astype(jnp.bfloat16).reshape(N, C, L)
    w1p = jnp.transpose(w1, (0, 2, 3, 1)).reshape(C, 9 * C)
    w2p = jnp.transpose(w2, (0, 2, 3, 1)).reshape(C, 9 * C)
    bb = jnp.stack([b1.astype(jnp.float32), b2.astype(jnp.float32)], axis=1)

    _body = functools.partial(_planar_kernel, B=B, H=H, W=W)

    out = pl.pallas_call(
        _body,
        out_shape=jax.ShapeDtypeStruct((N, C, L), jnp.float32),
        grid_spec=pltpu.PrefetchScalarGridSpec(
            num_scalar_prefetch=0,
            grid=(N // B,),
            in_specs=[
                pl.BlockSpec((B, C, L), lambda g: (g, 0, 0)),
                pl.BlockSpec((C, 9 * C), lambda g: (0, 0)),
                pl.BlockSpec((C, 9 * C), lambda g: (0, 0)),
                pl.BlockSpec((C, 2), lambda g: (0, 0)),
            ],
            out_specs=pl.BlockSpec((B, C, L), lambda g: (g, 0, 0)),
            scratch_shapes=[pltpu.VMEM((9 * C, L), jnp.bfloat16)],
        ),
        compiler_params=pltpu.CompilerParams(
            dimension_semantics=("parallel",),
            vmem_limit_bytes=64 * 1024 * 1024,
        ),
    )(xv, w1p.astype(jnp.bfloat16), w2p.astype(jnp.bfloat16), bb)

    return out.reshape(N, C, H, W)


# speedup vs baseline: 1.2051x; 1.2051x over previous
import functools

import jax
import jax.numpy as jnp
from jax.experimental import pallas as pl
from jax.experimental.pallas import tpu as pltpu


def _tap_stack(v, H, W, taps_ref):
    C, L = v.shape
    l = jax.lax.broadcasted_iota(jnp.int32, (1, L), 1)
    wpos = jax.lax.rem(l, W)
    for kh in range(3):
        for kw in range(3):
            delta = W * (kh - 1) + (kw - 1)
            t = jnp.roll(v, -delta, axis=1) if delta else v
            mask = None
            if kw == 0:
                mask = wpos != 0
            elif kw == 2:
                mask = wpos != W - 1
            if kh == 0:
                mh = l >= W
                mask = mh if mask is None else (mask & mh)
            elif kh == 2:
                mh = l < L - W
                mask = mh if mask is None else (mask & mh)
            if mask is not None:
                t = jnp.where(mask, t, jnp.bfloat16(0))
            taps_ref[C * (3 * kh + kw):C * (3 * kh + kw + 1), :] = t


def _planar_kernel(x_ref, w1_ref, w2_ref, b_ref, o_ref, ta_ref, tb_ref,
                   h_ref, *, B, H, W):
    C = x_ref.shape[1]
    b1c = b_ref[:, 0:1]
    b2c = b_ref[:, 1:2]
    w1v = w1_ref[...]
    w2v = w2_ref[...]
    for b in range(B):
        t_ref = ta_ref if b % 2 == 0 else tb_ref
        xb = x_ref[b].astype(jnp.bfloat16)
        _tap_stack(xb, H, W, t_ref)
        a1 = jnp.dot(w1v, t_ref[...], preferred_element_type=jnp.float32)
        h_ref[C * b:C * (b + 1), :] = jnp.maximum(a1 + b1c,
                                                  0.0).astype(jnp.bfloat16)
    for b in range(B):
        t_ref = ta_ref if b % 2 == 0 else tb_ref
        _tap_stack(h_ref[C * b:C * (b + 1), :], H, W, t_ref)
        a2 = jnp.dot(w2v, t_ref[...], preferred_element_type=jnp.float32)
        o_ref[b] = jnp.maximum(a2 + b2c, 0.0)


def kernel(x_nchw, w1, b1, w2, b2):
    N, C, H, W = x_nchw.shape
    B = 16 if N % 16 == 0 else 8
    L = H * W

    xv = x_nchw.reshape(N, C, L)
    w1p = jnp.transpose(w1, (0, 2, 3, 1)).reshape(C, 9 * C)
    w2p = jnp.transpose(w2, (0, 2, 3, 1)).reshape(C, 9 * C)
    bb = jnp.stack([b1.astype(jnp.float32), b2.astype(jnp.float32)], axis=1)

    _body = functools.partial(_planar_kernel, B=B, H=H, W=W)

    out = pl.pallas_call(
        _body,
        out_shape=jax.ShapeDtypeStruct((N, C, L), jnp.float32),
        grid_spec=pltpu.PrefetchScalarGridSpec(
            num_scalar_prefetch=0,
            grid=(N // B,),
            in_specs=[
                pl.BlockSpec((B, C, L), lambda g: (g, 0, 0)),
                pl.BlockSpec((C, 9 * C), lambda g: (0, 0)),
                pl.BlockSpec((C, 9 * C), lambda g: (0, 0)),
                pl.BlockSpec((C, 2), lambda g: (0, 0)),
            ],
            out_specs=pl.BlockSpec((B, C, L), lambda g: (g, 0, 0)),
            scratch_shapes=[pltpu.VMEM((9 * C, L), jnp.bfloat16),
                            pltpu.VMEM((9 * C, L), jnp.bfloat16),
                            pltpu.VMEM((B * C, L), jnp.bfloat16)],
        ),
        compiler_params=pltpu.CompilerParams(
            dimension_semantics=("parallel",),
            vmem_limit_bytes=64 * 1024 * 1024,
        ),
    )(xv, w1p.astype(jnp.bfloat16), w2p.astype(jnp.bfloat16), bb)

    return out.reshape(N, C, H, W)
